# baseline (speedup 1.0000x reference)
import functools

import jax
import jax.numpy as jnp
from jax import lax
from jax.experimental import pallas as pl
from jax.experimental.pallas import tpu as pltpu
from jax.experimental.pallas import tpu_sc as plsc

_NC = 2
_NS = 16
_NW = _NC * _NS

_GROUP = 128


@functools.cache
def _build(B, F, D):
    b_per_w = B // _NW
    g_per_f = b_per_w // _GROUP
    mesh = plsc.VectorSubcoreMesh(core_axis_name="c", subcore_axis_name="s")

    @functools.partial(
        pl.kernel,
        mesh=mesh,
        out_type=jax.ShapeDtypeStruct((F, D, B), jnp.float32),
        compiler_params=pltpu.CompilerParams(
            use_tc_tiling_on_sc=False, needs_layout_passes=False),
        scratch_types=[
            pltpu.VMEM((F, b_per_w), jnp.int32),
            pltpu.VMEM((F, b_per_w), jnp.float32),
            pltpu.VMEM((_GROUP,), jnp.int32),
            pltpu.VMEM((_GROUP, 128), jnp.float32),
            pltpu.VMEM((D, _GROUP), jnp.float32),
            pltpu.SemaphoreType.DMA,
        ],
    )
    def k(idx_hbm, val_hbm, tsup_hbm, out_hbm,
          idx_v, val_v, sidx_v, srows_v, outb_v, sem):
        wid = lax.axis_index("s") * _NC + lax.axis_index("c")
        b0 = wid * b_per_w
        pltpu.sync_copy(idx_hbm.at[:, pl.ds(b0, b_per_w)], idx_v)
        pltpu.sync_copy(val_hbm.at[:, pl.ds(b0, b_per_w)], val_v)
        lane = lax.iota(jnp.int32, 16)

        def group_body(fg, carry):
            f = fg // g_per_f
            gb = (fg % g_per_f) * _GROUP
            for j in range(_GROUP // 16):
                i16 = idx_v[f, pl.ds(gb + j * 16, 16)]
                sidx_v[pl.ds(j * 16, 16)] = lax.shift_right_logical(i16, 2)
            cp = pltpu.async_copy(tsup_hbm.at[sidx_v], srows_v, sem)
            cp.wait()
            for j in range(_GROUP // 16):
                i16 = idx_v[f, pl.ds(gb + j * 16, 16)]
                v16 = val_v[f, pl.ds(gb + j * 16, 16)]
                rows16 = lane + j * 16
                cols16 = lax.shift_left((i16 & 3), 5)
                for d in range(D):
                    o = plsc.load_gather(srows_v, [rows16, cols16 + d])
                    outb_v[d, pl.ds(j * 16, 16)] = o * v16
            pltpu.sync_copy(
                outb_v, out_hbm.at[f, :, pl.ds(b0 + gb, _GROUP)])
            return carry

        lax.fori_loop(0, F * g_per_f, group_body, 0)

    return k


def kernel(cat_index, cat_val, field_size, table):
    B, F = cat_index.shape
    V, D = table.shape
    tsup = table.reshape(V * D // 128, 128)
    idx_t = cat_index.T.astype(jnp.int32)
    val_t = cat_val.T
    out_t = _build(B, F, D)(idx_t, val_t, tsup)
    return out_t.transpose(2, 0, 1)


# speedup vs baseline: 1.1197x; 1.1197x over previous
import functools

import jax
import jax.numpy as jnp
from jax import lax
from jax.experimental import pallas as pl
from jax.experimental.pallas import tpu as pltpu
from jax.experimental.pallas import tpu_sc as plsc

_NC = 2
_NS = 16
_NW = _NC * _NS

_GROUP = 128


@functools.cache
def _build(B, F, D):
    b_per_w = B // _NW
    g_per_f = b_per_w // _GROUP
    mesh = plsc.VectorSubcoreMesh(core_axis_name="c", subcore_axis_name="s")

    @functools.partial(
        pl.kernel,
        mesh=mesh,
        out_type=jax.ShapeDtypeStruct((F, D, B), jnp.float32),
        compiler_params=pltpu.CompilerParams(
            use_tc_tiling_on_sc=False, needs_layout_passes=False),
        scratch_types=[
            pltpu.VMEM((F, b_per_w), jnp.int32),
            pltpu.VMEM((F, b_per_w), jnp.float32),
            pltpu.VMEM((b_per_w, D), jnp.float32),
            pltpu.VMEM((D, b_per_w), jnp.float32),
            pltpu.SemaphoreType.DMA,
        ],
    )
    def k(idx_hbm, val_hbm, table_hbm, out_hbm,
          idx_v, val_v, rows_v, outb_v, sem):
        wid = lax.axis_index("s") * _NC + lax.axis_index("c")
        b0 = wid * b_per_w
        pltpu.sync_copy(idx_hbm.at[:, pl.ds(b0, b_per_w)], idx_v)
        pltpu.sync_copy(val_hbm.at[:, pl.ds(b0, b_per_w)], val_v)
        lane = lax.iota(jnp.int32, 16)

        def field_body(f, carry):
            copies = [
                pltpu.async_copy(
                    table_hbm.at[idx_v.at[f, pl.ds(g * _GROUP, _GROUP)]],
                    rows_v.at[pl.ds(g * _GROUP, _GROUP)],
                    sem,
                )
                for g in range(g_per_f)
            ]
            for cp in copies:
                cp.wait()
            for j in range(b_per_w // 16):
                v16 = val_v[f, pl.ds(j * 16, 16)]
                rows16 = lane + j * 16
                for d in range(D):
                    o = plsc.load_gather(rows_v, [rows16, lane * 0 + d])
                    outb_v[d, pl.ds(j * 16, 16)] = o * v16
            pltpu.sync_copy(outb_v, out_hbm.at[f, :, pl.ds(b0, b_per_w)])
            return carry

        lax.fori_loop(0, F, field_body, 0)

    return k


def kernel(cat_index, cat_val, field_size, table):
    B, F = cat_index.shape
    V, D = table.shape
    idx_t = cat_index.T.astype(jnp.int32)
    val_t = cat_val.T
    out_t = _build(B, F, D)(idx_t, val_t, table)
    return out_t.transpose(2, 0, 1)


# speedup vs baseline: 1.4674x; 1.3105x over previous
import functools

import jax
import jax.numpy as jnp
from jax import lax
from jax.experimental import pallas as pl
from jax.experimental.pallas import tpu as pltpu
from jax.experimental.pallas import tpu_sc as plsc

_NC = 2
_NS = 16
_NW = _NC * _NS

_GROUP = 128
_TCC = 1024


@functools.cache
def _build_tconv(V, D):
    def body(t_ref, o_ref):
        o_ref[...] = t_ref[...].T.reshape(_TCC * D // 128, 128)

    return pl.pallas_call(
        body,
        grid=(pl.cdiv(V, _TCC),),
        in_specs=[pl.BlockSpec((D, _TCC), lambda g: (0, g))],
        out_specs=pl.BlockSpec((_TCC * D // 128, 128), lambda g: (g, 0)),
        out_shape=jax.ShapeDtypeStruct((V * D // 128, 128), jnp.float32),
    )


@functools.cache
def _build(B, F, D):
    b_per_w = B // _NW
    g_per_f = b_per_w // _GROUP
    stride = b_per_w + 1
    mesh = plsc.VectorSubcoreMesh(core_axis_name="c", subcore_axis_name="s")

    @functools.partial(
        pl.kernel,
        mesh=mesh,
        out_type=jax.ShapeDtypeStruct((F, D, B), jnp.float32),
        compiler_params=pltpu.CompilerParams(
            use_tc_tiling_on_sc=False, needs_layout_passes=False),
        scratch_types=[
            pltpu.VMEM((F, b_per_w), jnp.int32),
            pltpu.VMEM((F, b_per_w), jnp.float32),
            pltpu.VMEM((b_per_w, D), jnp.float32),
            pltpu.VMEM((D, stride), jnp.float32),
            pltpu.SemaphoreType.DMA,
        ],
    )
    def k(idx_hbm, val_hbm, table_hbm, out_hbm,
          idx_v, val_v, rows_v, outb_v, sem):
        wid = lax.axis_index("s") * _NC + lax.axis_index("c")
        b0 = wid * b_per_w
        pltpu.sync_copy(idx_hbm.at[:, pl.ds(b0, b_per_w)], idx_v)
        pltpu.sync_copy(val_hbm.at[:, pl.ds(b0, b_per_w)], val_v)
        lane = lax.iota(jnp.int32, 16)
        d_lo = lane
        d_hi = lane + 16

        def field_body(f, carry):
            copies = [
                pltpu.async_copy(
                    table_hbm.at[idx_v.at[f, pl.ds(g * _GROUP, _GROUP)]],
                    rows_v.at[pl.ds(g * _GROUP, _GROUP)],
                    sem,
                )
                for g in range(g_per_f)
            ]
            for cp in copies:
                cp.wait()

            def j_body(j, carry2):
                vvec = val_v[f, pl.ds(j * 16, 16)]
                for u in range(16):
                    r = j * 16 + u
                    v = vvec[u]
                    rvec = lane * 0 + r
                    plsc.store_scatter(
                        outb_v, [d_lo, rvec],
                        rows_v[r, pl.ds(0, 16)] * v)
                    plsc.store_scatter(
                        outb_v, [d_hi, rvec],
                        rows_v[r, pl.ds(16, 16)] * v)
                return carry2

            lax.fori_loop(0, b_per_w // 16, j_body, 0)
            pltpu.sync_copy(
                outb_v.at[:, pl.ds(0, b_per_w)],
                out_hbm.at[f, :, pl.ds(b0, b_per_w)])
            return carry

        lax.fori_loop(0, F, field_body, 0)

    return k


def kernel(cat_index, cat_val, field_size, table):
    B, F = cat_index.shape
    V, D = table.shape
    idx_t = cat_index.T.astype(jnp.int32)
    val_t = cat_val.T
    out_t = _build(B, F, D)(idx_t, val_t, table)
    return out_t.transpose(2, 0, 1)


# speedup vs baseline: 1.5663x; 1.0674x over previous
import functools

import jax
import jax.numpy as jnp
from jax import lax
from jax.experimental import pallas as pl
from jax.experimental.pallas import tpu as pltpu
from jax.experimental.pallas import tpu_sc as plsc

_NC = 2
_NS = 16
_NW = _NC * _NS

_GROUP = 128
_TCC = 1024


@functools.cache
def _build_tconv(V, D):
    def body(t_ref, o_ref):
        o_ref[...] = t_ref[...].T.reshape(_TCC * D // 128, 128)

    return pl.pallas_call(
        body,
        grid=(pl.cdiv(V, _TCC),),
        in_specs=[pl.BlockSpec((D, _TCC), lambda g: (0, g))],
        out_specs=pl.BlockSpec((_TCC * D // 128, 128), lambda g: (g, 0)),
        out_shape=jax.ShapeDtypeStruct((V * D // 128, 128), jnp.float32),
    )


@functools.cache
def _build(B, F, D):
    b_per_w = B // _NW
    g_per_f = b_per_w // _GROUP
    stride = b_per_w + 1
    mesh = plsc.VectorSubcoreMesh(core_axis_name="c", subcore_axis_name="s")

    @functools.partial(
        pl.kernel,
        mesh=mesh,
        out_type=jax.ShapeDtypeStruct((F, D, B), jnp.float32),
        compiler_params=pltpu.CompilerParams(
            use_tc_tiling_on_sc=False, needs_layout_passes=False),
        scratch_types=[
            pltpu.VMEM((F, b_per_w), jnp.int32),
            pltpu.VMEM((F, b_per_w), jnp.float32),
            pltpu.VMEM((b_per_w, D), jnp.float32),
            pltpu.VMEM((b_per_w, D), jnp.float32),
            pltpu.VMEM((D, stride), jnp.float32),
            pltpu.VMEM((D, stride), jnp.float32),
            pltpu.SemaphoreType.DMA,
            pltpu.SemaphoreType.DMA,
            pltpu.SemaphoreType.DMA,
        ],
    )
    def k(idx_hbm, val_hbm, table_hbm, out_hbm,
          idx_v, val_v, rows0_v, rows1_v, outb0_v, outb1_v,
          gsem0, gsem1, osem):
        wid = lax.axis_index("s") * _NC + lax.axis_index("c")
        b0 = wid * b_per_w
        pltpu.sync_copy(idx_hbm.at[:, pl.ds(b0, b_per_w)], idx_v)
        pltpu.sync_copy(val_hbm.at[:, pl.ds(b0, b_per_w)], val_v)
        lane = lax.iota(jnp.int32, 16)
        d_lo = lane
        d_hi = lane + 16
        rows_bufs = [rows0_v, rows1_v]
        out_bufs = [outb0_v, outb1_v]
        g_sems = [gsem0, gsem1]

        def fire(f, p):
            for g in range(g_per_f):
                pltpu.async_copy(
                    table_hbm.at[idx_v.at[f, pl.ds(g * _GROUP, _GROUP)]],
                    rows_bufs[p].at[pl.ds(g * _GROUP, _GROUP)],
                    g_sems[p],
                )

        def drain_gather(p):
            pltpu.make_async_copy(
                table_hbm.at[pl.ds(0, b_per_w)], rows_bufs[p], g_sems[p],
            ).wait()

        def drain_out(p):
            pltpu.make_async_copy(
                out_hbm.at[0, :, pl.ds(0, b_per_w)],
                out_bufs[p].at[:, pl.ds(0, b_per_w)],
                osem,
            ).wait()

        fire(0, 0)

        def field_body(f, carry):
            for p in (0, 1):

                @pl.when(lax.rem(f, 2) == p)
                def _(p=p):
                    drain_gather(p)

                    @pl.when(f + 1 < F)
                    def _():
                        fire(f + 1, 1 - p)

                    @pl.when(f >= 2)
                    def _():
                        drain_out(p)

                    rows_v = rows_bufs[p]
                    outb_v = out_bufs[p]

                    def j_body(j, carry2):
                        vvec = val_v[f, pl.ds(j * 16, 16)]
                        for u in range(16):
                            r = j * 16 + u
                            v = vvec[u]
                            rvec = lane * 0 + r
                            plsc.store_scatter(
                                outb_v, [d_lo, rvec],
                                rows_v[r, pl.ds(0, 16)] * v)
                            plsc.store_scatter(
                                outb_v, [d_hi, rvec],
                                rows_v[r, pl.ds(16, 16)] * v)
                        return carry2

                    lax.fori_loop(0, b_per_w // 16, j_body, 0)
                    pltpu.async_copy(
                        outb_v.at[:, pl.ds(0, b_per_w)],
                        out_hbm.at[f, :, pl.ds(b0, b_per_w)],
                        osem)
            return carry

        lax.fori_loop(0, F, field_body, 0)
        drain_out(0)
        drain_out(1)

    return k


def kernel(cat_index, cat_val, field_size, table):
    B, F = cat_index.shape
    V, D = table.shape
    idx_t = cat_index.T.astype(jnp.int32)
    val_t = cat_val.T
    out_t = _build(B, F, D)(idx_t, val_t, table)
    return out_t.transpose(2, 0, 1)
